# Initial kernel scaffold; baseline (speedup 1.0000x reference)
import jax
import jax.numpy as jnp
from jax.experimental import pallas as pl


def kernel(x, edge_index, edge_index_b, lam, id_new_value_old, edge_weight, W0, b0, W1, b1, W2, b2, W3, b3, clsW, clsb):
    raise NotImplementedError("write your pallas kernel here")


# speedup vs baseline: 7.4548x; 7.4548x over previous
import functools

import jax
import jax.numpy as jnp
from jax import lax
from jax.experimental import pallas as pl
from jax.experimental.pallas import tpu as pltpu
from jax.experimental.pallas import tpu_sc as plsc

N = 10000
E = 160000
H = 256
HH = 128
C = 64
RW = 0.8
NC = 2
NS = 16

f32 = jnp.float32
i32 = jnp.int32

_MESH = plsc.VectorSubcoreMesh(core_axis_name="c", subcore_axis_name="s",
                               num_cores=NC, num_subcores=NS)

EPW = E // (NC * NS)


@functools.partial(
    pl.kernel,
    out_type=(
        jax.ShapeDtypeStruct((2 * E,), i32),
        jax.ShapeDtypeStruct((2 * E,), i32),
        jax.ShapeDtypeStruct((E,), f32),
    ),
    mesh=_MESH,
    scratch_types=(
        pltpu.VMEM((EPW,), i32),
        pltpu.VMEM((EPW,), i32),
        pltpu.VMEM((EPW + 16,), f32),
        pltpu.SemaphoreType.DMA,
    ),
)
def _prep(ei, eib, idvo, ew, srcs, dsts, wsc, a_v, b_v, w_v, sem):
    c = lax.axis_index("c")
    s = lax.axis_index("s")
    wid = s * NC + c
    e0 = wid * EPW
    pltpu.sync_copy(ei.at[pl.ds(e0, EPW)], a_v)
    pltpu.sync_copy(a_v, srcs.at[pl.ds(e0, EPW)])
    pltpu.sync_copy(ei.at[pl.ds(E + e0, EPW)], a_v)
    pltpu.sync_copy(a_v, dsts.at[pl.ds(e0, EPW)])
    pltpu.sync_copy(eib.at[pl.ds(e0, EPW)], a_v)
    pltpu.async_copy(idvo.at[a_v], b_v, sem).wait()
    pltpu.sync_copy(b_v, srcs.at[pl.ds(E + e0, EPW)])
    pltpu.sync_copy(eib.at[pl.ds(E + e0, EPW)], a_v)
    pltpu.sync_copy(a_v, dsts.at[pl.ds(E + e0, EPW)])
    pltpu.sync_copy(ew.at[pl.ds(e0, EPW)], w_v.at[pl.ds(0, EPW)])

    def _scale(i, carry):
        w_v[pl.ds(i * 16, 16)] = w_v[pl.ds(i * 16, 16)] * RW
        return carry

    lax.fori_loop(0, (EPW + 15) // 16, _scale, 0)
    pltpu.sync_copy(w_v.at[pl.ds(0, EPW)], wsc.at[pl.ds(e0, EPW)])


RPW = 312
RREM = N - RPW * NC * NS


@functools.partial(
    pl.kernel,
    out_type=(
        jax.ShapeDtypeStruct((N, HH), f32),
        jax.ShapeDtypeStruct((N, HH), f32),
    ),
    mesh=_MESH,
    scratch_types=(
        pltpu.VMEM((RPW,), i32),
        pltpu.VMEM((RPW, HH), f32),
        pltpu.SemaphoreType.DMA,
    ),
)
def _permute_rows(vh0, vh1, idvo, o0, o1, idx_v, rows_v, sem):
    c = lax.axis_index("c")
    s = lax.axis_index("s")
    wid = s * NC + c
    r0 = wid * RPW
    pltpu.sync_copy(idvo.at[pl.ds(r0, RPW)], idx_v)
    pltpu.async_copy(vh0.at[idx_v], rows_v, sem).wait()
    pltpu.sync_copy(rows_v, o0.at[pl.ds(r0, RPW)])
    pltpu.async_copy(vh1.at[idx_v], rows_v, sem).wait()
    pltpu.sync_copy(rows_v, o1.at[pl.ds(r0, RPW)])

    @pl.when(wid == 0)
    def _tail():
        base = RPW * NC * NS
        idx_t = idx_v.at[pl.ds(0, RREM)]
        rows_t = rows_v.at[pl.ds(0, RREM)]
        pltpu.sync_copy(idvo.at[pl.ds(base, RREM)], idx_t)
        pltpu.async_copy(vh0.at[idx_t], rows_t, sem).wait()
        pltpu.sync_copy(rows_t, o0.at[pl.ds(base, RREM)])
        pltpu.async_copy(vh1.at[idx_t], rows_t, sem).wait()
        pltpu.sync_copy(rows_t, o1.at[pl.ds(base, RREM)])


EPT = E // NS
CH = 128
NCHUNK = EPT // CH
NTRIP = NCHUNK // 3
ETAIL = EPT - NCHUNK * CH
RPT = 624
RTAIL = N - NS * RPT


@functools.partial(
    pl.kernel,
    out_type=(
        jax.ShapeDtypeStruct((2, N, HH), f32),
        jax.ShapeDtypeStruct((2, N, HH), f32),
    ),
    mesh=_MESH,
    scratch_types=(
        pltpu.VMEM_SHARED((N, HH), f32),
        pltpu.VMEM((CH,), i32), pltpu.VMEM((CH,), i32), pltpu.VMEM((CH,), i32),
        pltpu.VMEM((CH,), i32), pltpu.VMEM((CH,), i32), pltpu.VMEM((CH,), i32),
        pltpu.VMEM((CH,), f32), pltpu.VMEM((CH,), f32), pltpu.VMEM((CH,), f32),
        pltpu.VMEM((CH, HH), f32), pltpu.VMEM((CH, HH), f32),
        pltpu.VMEM((CH, HH), f32),
        pltpu.SemaphoreType.DMA, pltpu.SemaphoreType.DMA,
        pltpu.SemaphoreType.DMA,
        pltpu.SemaphoreType.DMA, pltpu.SemaphoreType.DMA,
        pltpu.SemaphoreType.DMA,
        pltpu.SemaphoreType.DMA, pltpu.SemaphoreType.DMA,
        pltpu.SemaphoreType.DMA,
    ),
)
def _agg(vh0, vh1, srcs, dsts, wsc, aggh0, aggh1,
         acc, s0, s1, s2, d0, d1, d2, wv0, wv1, wv2, r0b, r1b, r2b,
         sg0, sg1, sg2, ss0, ss1, ss2, si0, si1, si2):
    c = lax.axis_index("c")
    s = lax.axis_index("s")
    zvec = jnp.zeros((16,), f32)
    row0 = s * RPT
    srcb = (s0, s1, s2)
    dstb = (d0, d1, d2)
    wb = (wv0, wv1, wv2)
    rows = (r0b, r1b, r2b)
    sg = (sg0, sg1, sg2)
    ss = (ss0, ss1, ss2)
    si = (si0, si1, si2)

    def _idx_async(ci, b):
        e0 = c * E + s * EPT + ci * CH
        pltpu.async_copy(srcs.at[pl.ds(e0, CH)], srcb[b], si[b])
        pltpu.async_copy(dsts.at[pl.ds(e0, CH)], dstb[b], si[b])
        pltpu.async_copy(wsc.at[pl.ds(s * EPT + ci * CH, CH)], wb[b], si[b])

    def _idx_drain(ci, b):
        e0 = c * E + s * EPT + ci * CH
        pltpu.make_async_copy(srcs.at[pl.ds(e0, CH)], srcb[b], si[b]).wait()
        pltpu.make_async_copy(dsts.at[pl.ds(e0, CH)], dstb[b], si[b]).wait()
        pltpu.make_async_copy(wsc.at[pl.ds(s * EPT + ci * CH, CH)],
                              wb[b], si[b]).wait()

    def _scale(b, nedge=CH):
        def _g16(g, nlane):
            wvec = wb[b][pl.ds(g * 16, 16)]
            for j in range(nlane):
                k = g * 16 + j
                for f in range(HH // 16):
                    rows[b][k, pl.ds(f * 16, 16)] = (
                        rows[b][k, pl.ds(f * 16, 16)] * wvec[j])

        def _grp(g, kc):
            _g16(g * 2, 16)
            _g16(g * 2 + 1, 16)
            return kc

        lax.fori_loop(0, nedge // 32, _grp, 0)
        if (nedge // 16) % 2:
            _g16(nedge // 16 - 1, 16)
        if nedge % 16:
            _g16(nedge // 16, nedge % 16)

    def _wait_scatter(b):
        pltpu.make_async_copy(rows[b], acc.at[dstb[b]], ss[b]).wait()

    for p in range(2):
        vh = vh0 if p == 0 else vh1
        aggh = aggh0 if p == 0 else aggh1

        def _zb(i, carry):
            for f in range(HH // 16):
                r0b[i, pl.ds(f * 16, 16)] = zvec
            return carry

        lax.fori_loop(0, CH, _zb, 0)
        for z in range(RPT // CH):
            pltpu.sync_copy(r0b, acc.at[pl.ds(row0 + z * CH, CH)])
        zrem = RPT - (RPT // CH) * CH
        if zrem:
            pltpu.sync_copy(r0b.at[pl.ds(0, zrem)],
                            acc.at[pl.ds(row0 + (RPT // CH) * CH, zrem)])

        @pl.when(s == 0)
        def _ztail():
            pltpu.sync_copy(r0b.at[pl.ds(0, RTAIL)],
                            acc.at[pl.ds(NS * RPT, RTAIL)])

        plsc.subcore_barrier()

        _idx_async(0, 0)
        _idx_async(1, 1)
        _idx_drain(0, 0)
        pltpu.async_copy(vh.at[srcb[0]], rows[0], sg[0])
        _idx_drain(1, 1)
        pltpu.async_copy(vh.at[srcb[1]], rows[1], sg[1])

        def _step(ct, b, first):
            ci = ct * 3 + b
            bn = (b + 2) % 3
            pltpu.make_async_copy(vh.at[srcb[b]], rows[b], sg[b]).wait()
            if not first:
                _wait_scatter(bn)

            @pl.when(ci + 2 < NCHUNK)
            def _pref_idx():
                _idx_async(ci + 2, bn)

            _scale(b)
            pltpu.async_copy(rows[b], acc.at[dstb[b]], ss[b], add=True)

            @pl.when(ci + 2 < NCHUNK)
            def _pref_gather():
                _idx_drain(ci + 2, bn)
                pltpu.async_copy(vh.at[srcb[bn]], rows[bn], sg[bn])

        _step(0, 0, True)
        _step(0, 1, False)
        _step(0, 2, False)

        def _trip(ct, carry):
            _step(ct, 0, False)
            _step(ct, 1, False)
            _step(ct, 2, False)
            return carry

        lax.fori_loop(1, NTRIP, _trip, 0)
        _wait_scatter((NCHUNK - 1) % 3)

        if ETAIL:
            et0 = c * E + s * EPT + NCHUNK * CH
            pltpu.sync_copy(srcs.at[pl.ds(et0, ETAIL)],
                            s0.at[pl.ds(0, ETAIL)])
            pltpu.sync_copy(dsts.at[pl.ds(et0, ETAIL)],
                            d0.at[pl.ds(0, ETAIL)])
            pltpu.sync_copy(wsc.at[pl.ds(s * EPT + NCHUNK * CH, ETAIL)],
                            wv0.at[pl.ds(0, ETAIL)])
            rows_t = r0b.at[pl.ds(0, ETAIL)]
            pltpu.async_copy(vh.at[s0.at[pl.ds(0, ETAIL)]],
                             rows_t, sg0).wait()
            _scale(0, ETAIL)
            pltpu.sync_copy(rows_t, acc.at[d0.at[pl.ds(0, ETAIL)]],
                            add=True)

        plsc.subcore_barrier()
        pltpu.sync_copy(acc.at[pl.ds(row0, RPT)],
                        aggh.at[c, pl.ds(row0, RPT)])

        @pl.when(s == 0)
        def _wtail():
            pltpu.sync_copy(acc.at[pl.ds(NS * RPT, RTAIL)],
                            aggh.at[c, pl.ds(NS * RPT, RTAIL)])

        plsc.subcore_barrier()


RB = 1000
GRID = N // RB


def _tc0_body(x_ref, w_ref, o0, o1):
    p = jnp.dot(x_ref[...], w_ref[...], preferred_element_type=f32)
    o0[...] = p[:, :HH]
    o1[...] = p[:, HH:]


def _tc0(x, W0):
    return pl.pallas_call(
        _tc0_body,
        grid=(GRID,),
        in_specs=[
            pl.BlockSpec((RB, H), lambda i: (i, 0)),
            pl.BlockSpec((H, H), lambda i: (0, 0)),
        ],
        out_specs=[
            pl.BlockSpec((RB, HH), lambda i: (i, 0)),
            pl.BlockSpec((RB, HH), lambda i: (i, 0)),
        ],
        out_shape=[jax.ShapeDtypeStruct((N, HH), f32)] * 2,
    )(x, W0)


def _tc1_body(h0, h1, p0, p1, a0, a1, b_ref, lam_ref, w_ref, q0, q1, m_ref):
    P = jnp.concatenate([h0[...], h1[...]], axis=1)
    Pp = jnp.concatenate([p0[...], p1[...]], axis=1)
    A0 = a0[...]
    A1 = a1[...]
    aA = jnp.concatenate([A0[0], A1[0]], axis=1)
    aB = jnp.concatenate([A0[1], A1[1]], axis=1)
    b = b_ref[0]
    lam = lam_ref[0, 0]
    x1 = jax.nn.relu(aA + P + b)
    xm = lam * P + (1.0 - lam) * Pp
    x1m = (lam * jax.nn.relu(aA + xm + b)
           + (1.0 - lam) * jax.nn.relu(aB + xm + b))
    q = jnp.dot(x1, w_ref[...], preferred_element_type=f32)
    m = jnp.dot(x1m, w_ref[...], preferred_element_type=f32)
    q0[...] = q[:, :HH]
    q1[...] = q[:, HH:]
    m_ref[...] = m


def _tc1(h0, h1, p0, p1, a0, a1, b, lam2, W):
    half = pl.BlockSpec((RB, HH), lambda i: (i, 0))
    return pl.pallas_call(
        _tc1_body,
        grid=(GRID,),
        in_specs=[
            half, half, half, half,
            pl.BlockSpec((2, RB, HH), lambda i: (0, i, 0)),
            pl.BlockSpec((2, RB, HH), lambda i: (0, i, 0)),
            pl.BlockSpec((1, H), lambda i: (0, 0)),
            pl.BlockSpec((1, 1), lambda i: (0, 0)),
            pl.BlockSpec((H, H), lambda i: (0, 0)),
        ],
        out_specs=[half, half, pl.BlockSpec((RB, H), lambda i: (i, 0))],
        out_shape=[
            jax.ShapeDtypeStruct((N, HH), f32),
            jax.ShapeDtypeStruct((N, HH), f32),
            jax.ShapeDtypeStruct((N, H), f32),
        ],
    )(h0, h1, p0, p1, a0, a1, b, lam2, W)


def _tcmid_body(h0, h1, mp, a0, a1, b_ref, lam_ref, w_ref, q0, q1, m_ref):
    Q = jnp.concatenate([h0[...], h1[...]], axis=1)
    A0 = a0[...]
    A1 = a1[...]
    aA = jnp.concatenate([A0[0], A1[0]], axis=1)
    aB = jnp.concatenate([A0[1], A1[1]], axis=1)
    b = b_ref[0]
    lam = lam_ref[0, 0]
    M = mp[...]
    hn = jax.nn.relu(aA + Q + b)
    hm = (lam * jax.nn.relu(aA + M + b)
          + (1.0 - lam) * jax.nn.relu(aB + M + b))
    q = jnp.dot(hn, w_ref[...], preferred_element_type=f32)
    m = jnp.dot(hm, w_ref[...], preferred_element_type=f32)
    q0[...] = q[:, :HH]
    q1[...] = q[:, HH:]
    m_ref[...] = m


def _tcmid(h0, h1, mp, a0, a1, b, lam2, W):
    half = pl.BlockSpec((RB, HH), lambda i: (i, 0))
    return pl.pallas_call(
        _tcmid_body,
        grid=(GRID,),
        in_specs=[
            half, half,
            pl.BlockSpec((RB, H), lambda i: (i, 0)),
            pl.BlockSpec((2, RB, HH), lambda i: (0, i, 0)),
            pl.BlockSpec((2, RB, HH), lambda i: (0, i, 0)),
            pl.BlockSpec((1, H), lambda i: (0, 0)),
            pl.BlockSpec((1, 1), lambda i: (0, 0)),
            pl.BlockSpec((H, H), lambda i: (0, 0)),
        ],
        out_specs=[half, half, pl.BlockSpec((RB, H), lambda i: (i, 0))],
        out_shape=[
            jax.ShapeDtypeStruct((N, HH), f32),
            jax.ShapeDtypeStruct((N, HH), f32),
            jax.ShapeDtypeStruct((N, H), f32),
        ],
    )(h0, h1, mp, a0, a1, b, lam2, W)


def _tc4_body(mp, a0, a1, b_ref, lam_ref, w_ref, cb_ref, o_ref):
    A0 = a0[...]
    A1 = a1[...]
    aA = jnp.concatenate([A0[0], A1[0]], axis=1)
    aB = jnp.concatenate([A0[1], A1[1]], axis=1)
    b = b_ref[0]
    lam = lam_ref[0, 0]
    M = mp[...]
    hm = (lam * jax.nn.relu(aA + M + b)
          + (1.0 - lam) * jax.nn.relu(aB + M + b))
    o_ref[...] = jnp.dot(hm, w_ref[...], preferred_element_type=f32) + cb_ref[0]


def _tc4(mp, a0, a1, b, lam2, clsW, clsb):
    return pl.pallas_call(
        _tc4_body,
        grid=(GRID,),
        in_specs=[
            pl.BlockSpec((RB, H), lambda i: (i, 0)),
            pl.BlockSpec((2, RB, HH), lambda i: (0, i, 0)),
            pl.BlockSpec((2, RB, HH), lambda i: (0, i, 0)),
            pl.BlockSpec((1, H), lambda i: (0, 0)),
            pl.BlockSpec((1, 1), lambda i: (0, 0)),
            pl.BlockSpec((H, C), lambda i: (0, 0)),
            pl.BlockSpec((1, C), lambda i: (0, 0)),
        ],
        out_specs=pl.BlockSpec((RB, C), lambda i: (i, 0)),
        out_shape=jax.ShapeDtypeStruct((N, C), f32),
    )(mp, a0, a1, b, lam2, clsW, clsb)


def kernel(x, edge_index, edge_index_b, lam, id_new_value_old, edge_weight,
           W0, b0, W1, b1, W2, b2, W3, b3, clsW, clsb):
    ei = edge_index.astype(i32)
    eib = edge_index_b.astype(i32)
    idvo = id_new_value_old.astype(i32)
    lam2 = jnp.asarray(lam, f32).reshape(1, 1)

    srcs, dsts, wsc = _prep(ei.reshape(2 * E), eib.reshape(2 * E), idvo,
                            edge_weight.astype(f32))

    ph0, ph1 = _tc0(x, W0)
    pp0, pp1 = _permute_rows(ph0, ph1, idvo)
    a0h0, a0h1 = _agg(ph0, ph1, srcs, dsts, wsc)
    q0, q1, m1 = _tc1(ph0, ph1, pp0, pp1, a0h0, a0h1,
                      b0.reshape(1, H), lam2, W1)
    a1h0, a1h1 = _agg(q0, q1, srcs, dsts, wsc)
    r0, r1, m2 = _tcmid(q0, q1, m1, a1h0, a1h1, b1.reshape(1, H), lam2, W2)
    a2h0, a2h1 = _agg(r0, r1, srcs, dsts, wsc)
    s0, s1, m3 = _tcmid(r0, r1, m2, a2h0, a2h1, b2.reshape(1, H), lam2, W3)
    a3h0, a3h1 = _agg(s0, s1, srcs, dsts, wsc)
    return _tc4(m3, a3h0, a3h1, b3.reshape(1, H), lam2, clsW,
                clsb.reshape(1, C))
